# baseline (device time: 90713 ns/iter reference)
import jax
import jax.numpy as jnp
from jax import lax
from jax.experimental import pallas as pl
from jax.experimental.pallas import tpu as pltpu

N_DEV = 4
B, Sq, D = 2, 256, 512
Hq, Dh = 16, 64
H_LOC = Hq // N_DEV
HD_LOC = H_LOC * Dh
SKV_LOC = 256
SKV = SKV_LOC * N_DEV
DOUT = 512
BLK = 64
SCALE = 0.125
NEG = -1e9

_CompilerParams = getattr(pltpu, "CompilerParams", None) or pltpu.TPUCompilerParams


def kernel(x, Wq, K_ext, V_ext, Wo):
    def body(x_ref, wq_ref, k_ref, v_ref, wo_ref, out_ref,
             k_chunks, v_chunks, my_part, part_gather,
             send_k, send_v, send_p, recv_k, recv_v, recv_p):
        me = lax.axis_index("i")

        barrier = pltpu.get_barrier_semaphore()
        for d in range(1, N_DEV):
            pl.semaphore_signal(
                barrier, inc=1,
                device_id=((me + d) % N_DEV,),
                device_id_type=pl.DeviceIdType.MESH,
            )
        pl.semaphore_wait(barrier, N_DEV - 1)

        kv_rdmas = []
        for d in (2, 1, 3):
            peer = (me + d) % N_DEV
            slot = N_DEV - 1 - d
            for (src, chunks, ssem, rsem) in (
                (k_ref, k_chunks, send_k, recv_k),
                (v_ref, v_chunks, send_v, recv_v),
            ):
                rdma = pltpu.make_async_remote_copy(
                    src_ref=src.at[:, :, pl.ds(peer * H_LOC, H_LOC), :],
                    dst_ref=chunks.at[slot],
                    send_sem=ssem.at[d - 1],
                    recv_sem=rsem.at[slot],
                    device_id=(peer,),
                    device_id_type=pl.DeviceIdType.MESH,
                )
                rdma.start()
                kv_rdmas.append(rdma)

        q = jnp.dot(x_ref[...].reshape(B * Sq, D), wq_ref[...],
                    preferred_element_type=jnp.float32)

        qb = lax.broadcasted_iota(jnp.int32, (Sq, SKV_LOC), 0) // BLK
        jb = lax.broadcasted_iota(jnp.int32, (Sq, SKV_LOC), 1) // BLK

        m_s = [[jnp.full((Sq, 1), NEG, jnp.float32) for _ in range(H_LOC)]
               for _ in range(B)]
        l_s = [[jnp.zeros((Sq, 1), jnp.float32) for _ in range(H_LOC)]
               for _ in range(B)]
        acc = [[jnp.zeros((Sq, Dh), jnp.float32) for _ in range(H_LOC)]
               for _ in range(B)]

        def process_chunk(src_idx, kc, vc):
            kbb = src_idx * (SKV_LOC // BLK) + jb
            mask = (qb == kbb) | (kbb == 0) | (((qb + kbb) % 3) == 0)
            for b in range(B):
                kcb = kc[b].reshape(SKV_LOC, HD_LOC)
                vcb = vc[b].reshape(SKV_LOC, HD_LOC)
                for h in range(H_LOC):
                    qbh = lax.slice(q, (b * Sq, h * Dh),
                                    ((b + 1) * Sq, (h + 1) * Dh))
                    kbh = lax.slice(kcb, (0, h * Dh), (SKV_LOC, (h + 1) * Dh))
                    vbh = lax.slice(vcb, (0, h * Dh), (SKV_LOC, (h + 1) * Dh))
                    s = lax.dot_general(qbh, kbh, (((1,), (1,)), ((), ())),
                                        preferred_element_type=jnp.float32)
                    s = jnp.where(mask, s * SCALE, jnp.float32(NEG))
                    new_m = jnp.maximum(m_s[b][h],
                                        jnp.max(s, axis=1, keepdims=True))
                    p = jnp.exp(s - new_m)
                    corr = jnp.exp(m_s[b][h] - new_m)
                    l_s[b][h] = l_s[b][h] * corr + jnp.sum(p, axis=1,
                                                           keepdims=True)
                    acc[b][h] = acc[b][h] * corr + lax.dot_general(
                        p, vbh, (((1,), (0,)), ((), ())),
                        preferred_element_type=jnp.float32)
                    m_s[b][h] = new_m

        process_chunk(me,
                      k_ref[:, :, pl.ds(me * H_LOC, H_LOC), :],
                      v_ref[:, :, pl.ds(me * H_LOC, H_LOC), :])

        for d in (1, 3, 2):
            src = (me + d) % N_DEV
            slot = d - 1
            for (chunks, rsem) in ((k_chunks, recv_k), (v_chunks, recv_v)):
                pltpu.make_async_remote_copy(
                    src_ref=chunks.at[slot],
                    dst_ref=chunks.at[slot],
                    send_sem=rsem.at[slot],
                    recv_sem=rsem.at[slot],
                    device_id=(src,),
                    device_id_type=pl.DeviceIdType.MESH,
                ).wait_recv()
            process_chunk(src, k_chunks[slot], v_chunks[slot])

        parts = []
        for b in range(B):
            ctx_b = jnp.concatenate(
                [acc[b][h] / l_s[b][h] for h in range(H_LOC)], axis=1)
            parts.append(jnp.dot(ctx_b, wo_ref[...],
                                 preferred_element_type=jnp.float32))
        my_part[...] = jnp.stack(parts)

        p_rdmas = []
        for d in (2, 1, 3):
            peer = (me + d) % N_DEV
            slot = N_DEV - 1 - d
            rdma = pltpu.make_async_remote_copy(
                src_ref=my_part,
                dst_ref=part_gather.at[slot],
                send_sem=send_p.at[d - 1],
                recv_sem=recv_p.at[slot],
                device_id=(peer,),
                device_id_type=pl.DeviceIdType.MESH,
            )
            rdma.start()
            p_rdmas.append(rdma)

        for d in range(1, N_DEV):
            pltpu.make_async_remote_copy(
                src_ref=my_part,
                dst_ref=part_gather.at[d - 1],
                send_sem=recv_p.at[d - 1],
                recv_sem=recv_p.at[d - 1],
                device_id=((me + d) % N_DEV,),
                device_id_type=pl.DeviceIdType.MESH,
            ).wait_recv()

        acc_out = my_part[...]
        for j in range(N_DEV - 1):
            acc_out = acc_out + part_gather[j]
        out_ref[...] = acc_out

        for rdma in kv_rdmas + p_rdmas:
            rdma.wait_send()

    return pl.pallas_call(
        body,
        out_shape=jax.ShapeDtypeStruct((B, Sq, DOUT), jnp.float32),
        in_specs=[pl.BlockSpec(memory_space=pltpu.VMEM)] * 5,
        out_specs=pl.BlockSpec(memory_space=pltpu.VMEM),
        scratch_shapes=[
            pltpu.VMEM((N_DEV - 1, B, SKV_LOC, H_LOC, Dh), jnp.float32),
            pltpu.VMEM((N_DEV - 1, B, SKV_LOC, H_LOC, Dh), jnp.float32),
            pltpu.VMEM((B, Sq, DOUT), jnp.float32),
            pltpu.VMEM((N_DEV - 1, B, Sq, DOUT), jnp.float32),
            pltpu.SemaphoreType.DMA((N_DEV - 1,)),
            pltpu.SemaphoreType.DMA((N_DEV - 1,)),
            pltpu.SemaphoreType.DMA((N_DEV - 1,)),
            pltpu.SemaphoreType.DMA((N_DEV - 1,)),
            pltpu.SemaphoreType.DMA((N_DEV - 1,)),
            pltpu.SemaphoreType.DMA((N_DEV - 1,)),
        ],
        compiler_params=_CompilerParams(collective_id=0),
    )(x, Wq, K_ext, V_ext, Wo)


# device time: 44574 ns/iter; 2.0351x vs baseline; 2.0351x over previous
import jax
import jax.numpy as jnp
from jax import lax
from jax.experimental import pallas as pl
from jax.experimental.pallas import tpu as pltpu

N_DEV = 4
B, Sq, D = 2, 256, 512
Hq, Dh = 16, 64
H_LOC = Hq // N_DEV
HD_LOC = H_LOC * Dh
SKV_LOC = 256
SKV = SKV_LOC * N_DEV
DOUT = 512
BLK = 64
SCALE = 0.125
NEG = -1e9
BF16 = jnp.bfloat16
F32 = jnp.float32

_CompilerParams = getattr(pltpu, "CompilerParams", None) or pltpu.TPUCompilerParams


def kernel(x, Wq, K_ext, V_ext, Wo):
    def body(x_ref, wq_ref, k_ref, v_ref, wo_ref, out_ref,
             k_flat, v_flat, k_all, v_all, my_part, part_gather,
             send_k, send_v, send_p, recv_k, recv_v, recv_p):
        me = lax.axis_index("i")

        barrier = pltpu.get_barrier_semaphore()
        for d in range(1, N_DEV):
            pl.semaphore_signal(
                barrier, inc=1,
                device_id=((me + d) % N_DEV,),
                device_id_type=pl.DeviceIdType.MESH,
            )
        pl.semaphore_wait(barrier, N_DEV - 1)

        k_flat[...] = k_ref[...].astype(BF16).reshape(B, SKV_LOC, Hq * Dh)
        v_flat[...] = v_ref[...].astype(BF16).reshape(B, SKV_LOC, Hq * Dh)

        kv_rdmas = []
        for d in (2, 1, 3):
            peer = (me + d) % N_DEV
            slot = N_DEV - 1 - d
            for (flat, gathered, ssem, rsem) in (
                (k_flat, k_all, send_k, recv_k),
                (v_flat, v_all, send_v, recv_v),
            ):
                rdma = pltpu.make_async_remote_copy(
                    src_ref=flat.at[:, :, pl.ds(peer * HD_LOC, HD_LOC)],
                    dst_ref=gathered.at[:, pl.ds(me * SKV_LOC, SKV_LOC), :],
                    send_sem=ssem.at[d - 1],
                    recv_sem=rsem.at[slot],
                    device_id=(peer,),
                    device_id_type=pl.DeviceIdType.MESH,
                )
                rdma.start()
                kv_rdmas.append(rdma)

        k_all[:, pl.ds(me * SKV_LOC, SKV_LOC), :] = (
            k_flat[:, :, pl.ds(me * HD_LOC, HD_LOC)])
        v_all[:, pl.ds(me * SKV_LOC, SKV_LOC), :] = (
            v_flat[:, :, pl.ds(me * HD_LOC, HD_LOC)])

        q = jnp.dot(x_ref[...].reshape(B * Sq, D).astype(BF16),
                    wq_ref[...].astype(BF16),
                    preferred_element_type=F32).astype(BF16)

        for d in range(1, N_DEV):
            src = (me + d) % N_DEV
            for (gathered, rsem) in ((k_all, recv_k), (v_all, recv_v)):
                pltpu.make_async_remote_copy(
                    src_ref=gathered.at[:, pl.ds(src * SKV_LOC, SKV_LOC), :],
                    dst_ref=gathered.at[:, pl.ds(src * SKV_LOC, SKV_LOC), :],
                    send_sem=rsem.at[d - 1],
                    recv_sem=rsem.at[d - 1],
                    device_id=(src,),
                    device_id_type=pl.DeviceIdType.MESH,
                ).wait_recv()

        qb = lax.broadcasted_iota(jnp.int32, (Sq, SKV), 0) // BLK
        kb = lax.broadcasted_iota(jnp.int32, (Sq, SKV), 1) // BLK
        mask = (qb == kb) | (kb == 0) | (((qb + kb) % 3) == 0)

        wo16 = wo_ref[...].astype(BF16)
        parts = []
        for b in range(B):
            kb_all = k_all[b]
            vb_all = v_all[b]
            ctx_h = []
            for h in range(H_LOC):
                qbh = lax.slice(q, (b * Sq, h * Dh), ((b + 1) * Sq, (h + 1) * Dh))
                kbh = lax.slice(kb_all, (0, h * Dh), (SKV, (h + 1) * Dh))
                vbh = lax.slice(vb_all, (0, h * Dh), (SKV, (h + 1) * Dh))
                s = lax.dot_general(qbh, kbh, (((1,), (1,)), ((), ())),
                                    preferred_element_type=F32) * SCALE
                s = jnp.where(mask, s, jnp.float32(NEG))
                m = jnp.max(s, axis=1, keepdims=True)
                w = jnp.exp(s - m)
                w = w / jnp.sum(w, axis=1, keepdims=True)
                ctx_h.append(lax.dot_general(w.astype(BF16), vbh,
                                             (((1,), (0,)), ((), ())),
                                             preferred_element_type=F32))
            ctx_b = jnp.concatenate(ctx_h, axis=1)
            parts.append(jnp.dot(ctx_b.astype(BF16), wo16,
                                 preferred_element_type=F32))
        part_f32 = jnp.stack(parts)
        my_part[...] = part_f32.astype(BF16)

        p_rdmas = []
        for d in (2, 1, 3):
            peer = (me + d) % N_DEV
            slot = N_DEV - 1 - d
            rdma = pltpu.make_async_remote_copy(
                src_ref=my_part,
                dst_ref=part_gather.at[slot],
                send_sem=send_p.at[d - 1],
                recv_sem=recv_p.at[slot],
                device_id=(peer,),
                device_id_type=pl.DeviceIdType.MESH,
            )
            rdma.start()
            p_rdmas.append(rdma)

        for d in range(1, N_DEV):
            pltpu.make_async_remote_copy(
                src_ref=my_part,
                dst_ref=part_gather.at[d - 1],
                send_sem=recv_p.at[d - 1],
                recv_sem=recv_p.at[d - 1],
                device_id=((me + d) % N_DEV,),
                device_id_type=pl.DeviceIdType.MESH,
            ).wait_recv()

        acc = part_f32
        for j in range(N_DEV - 1):
            acc = acc + part_gather[j].astype(F32)
        out_ref[...] = acc

        for rdma in kv_rdmas + p_rdmas:
            rdma.wait_send()

    return pl.pallas_call(
        body,
        out_shape=jax.ShapeDtypeStruct((B, Sq, DOUT), F32),
        in_specs=[pl.BlockSpec(memory_space=pltpu.VMEM)] * 5,
        out_specs=pl.BlockSpec(memory_space=pltpu.VMEM),
        scratch_shapes=[
            pltpu.VMEM((B, SKV_LOC, Hq * Dh), BF16),
            pltpu.VMEM((B, SKV_LOC, Hq * Dh), BF16),
            pltpu.VMEM((B, SKV, HD_LOC), BF16),
            pltpu.VMEM((B, SKV, HD_LOC), BF16),
            pltpu.VMEM((B, Sq, DOUT), BF16),
            pltpu.VMEM((N_DEV - 1, B, Sq, DOUT), BF16),
            pltpu.SemaphoreType.DMA((N_DEV - 1,)),
            pltpu.SemaphoreType.DMA((N_DEV - 1,)),
            pltpu.SemaphoreType.DMA((N_DEV - 1,)),
            pltpu.SemaphoreType.DMA((N_DEV - 1,)),
            pltpu.SemaphoreType.DMA((N_DEV - 1,)),
            pltpu.SemaphoreType.DMA((N_DEV - 1,)),
        ],
        compiler_params=_CompilerParams(collective_id=0),
    )(x, Wq, K_ext, V_ext, Wo)


# device time: 43270 ns/iter; 2.0964x vs baseline; 1.0301x over previous
import jax
import jax.numpy as jnp
from jax import lax
from jax.experimental import pallas as pl
from jax.experimental.pallas import tpu as pltpu

N_DEV = 4
B, Sq, D = 2, 256, 512
Hq, Dh = 16, 64
H_LOC = Hq // N_DEV
HD_LOC = H_LOC * Dh
SKV_LOC = 256
SKV = SKV_LOC * N_DEV
DOUT = 512
BLK = 64
SCALE = 0.125
NEG = -1e9
BF16 = jnp.bfloat16
F32 = jnp.float32

_OFFSET_OF_BLOCK = (0, 1, 3, 2)
_BLOCK_OF_SEND = {1: 2, 2: 3, 3: 1}

_CompilerParams = getattr(pltpu, "CompilerParams", None) or pltpu.TPUCompilerParams


def kernel(x, Wq, K_ext, V_ext, Wo):
    def body(x_ref, wq_ref, k_ref, v_ref, wo_ref, out_ref,
             k_flat, v_flat, k_all, v_all, my_part, part_gather,
             send_k, send_v, send_p, recv_k, recv_v, recv_p):
        me = lax.axis_index("i")

        barrier = pltpu.get_barrier_semaphore()
        for d in range(1, N_DEV):
            pl.semaphore_signal(
                barrier, inc=1,
                device_id=((me + d) % N_DEV,),
                device_id_type=pl.DeviceIdType.MESH,
            )
        pl.semaphore_wait(barrier, N_DEV - 1)

        k_flat[...] = k_ref[...].astype(BF16).reshape(B, SKV_LOC, Hq * Dh)
        v_flat[...] = v_ref[...].astype(BF16).reshape(B, SKV_LOC, Hq * Dh)

        kv_rdmas = []
        for d in (2, 1, 3):
            peer = (me + d) % N_DEV
            blk = _BLOCK_OF_SEND[d]
            for (flat, gathered, ssem, rsem) in (
                (k_flat, k_all, send_k, recv_k),
                (v_flat, v_all, send_v, recv_v),
            ):
                rdma = pltpu.make_async_remote_copy(
                    src_ref=flat.at[:, :, pl.ds(peer * HD_LOC, HD_LOC)],
                    dst_ref=gathered.at[:, pl.ds(blk * SKV_LOC, SKV_LOC), :],
                    send_sem=ssem.at[d - 1],
                    recv_sem=rsem.at[blk - 1],
                    device_id=(peer,),
                    device_id_type=pl.DeviceIdType.MESH,
                )
                rdma.start()
                kv_rdmas.append(rdma)

        k_all[:, 0:SKV_LOC, :] = k_flat[:, :, pl.ds(me * HD_LOC, HD_LOC)]
        v_all[:, 0:SKV_LOC, :] = v_flat[:, :, pl.ds(me * HD_LOC, HD_LOC)]

        q = jnp.dot(x_ref[...].reshape(B * Sq, D).astype(BF16),
                    wq_ref[...].astype(BF16),
                    preferred_element_type=F32).astype(BF16)

        qb = lax.broadcasted_iota(jnp.int32, (Sq, SKV_LOC), 0) // BLK
        jb = lax.broadcasted_iota(jnp.int32, (Sq, SKV_LOC), 1) // BLK
        s_blocks = [[[] for _ in range(H_LOC)] for _ in range(B)]

        def score_block(blk):
            src = (me + _OFFSET_OF_BLOCK[blk]) % N_DEV
            kbb = src * (SKV_LOC // BLK) + jb
            mask = (qb == kbb) | (kbb == 0) | (((qb + kbb) % 3) == 0)
            for b in range(B):
                kcb = k_all[b, pl.ds(blk * SKV_LOC, SKV_LOC), :]
                for h in range(H_LOC):
                    qbh = lax.slice(q, (b * Sq, h * Dh),
                                    ((b + 1) * Sq, (h + 1) * Dh))
                    kbh = lax.slice(kcb, (0, h * Dh), (SKV_LOC, (h + 1) * Dh))
                    s = lax.dot_general(qbh, kbh, (((1,), (1,)), ((), ())),
                                        preferred_element_type=F32)
                    s_blocks[b][h].append(
                        jnp.where(mask, s * SCALE, jnp.float32(NEG)))

        score_block(0)
        for blk in (1, 2, 3):
            for (gathered, rsem) in ((k_all, recv_k), (v_all, recv_v)):
                pltpu.make_async_remote_copy(
                    src_ref=gathered.at[:, pl.ds(blk * SKV_LOC, SKV_LOC), :],
                    dst_ref=gathered.at[:, pl.ds(blk * SKV_LOC, SKV_LOC), :],
                    send_sem=rsem.at[blk - 1],
                    recv_sem=rsem.at[blk - 1],
                    device_id=((me + _OFFSET_OF_BLOCK[blk]) % N_DEV,),
                    device_id_type=pl.DeviceIdType.MESH,
                ).wait_recv()
            score_block(blk)

        wo16 = wo_ref[...].astype(BF16)
        p_rdmas = []
        parts_f32 = []
        for b in range(B):
            vb_all = v_all[b]
            ctx_h = []
            for h in range(H_LOC):
                s = jnp.concatenate(s_blocks[b][h], axis=1)
                m = jnp.max(s, axis=1, keepdims=True)
                w = jnp.exp(s - m)
                w = w / jnp.sum(w, axis=1, keepdims=True)
                vbh = lax.slice(vb_all, (0, h * Dh), (SKV, (h + 1) * Dh))
                ctx_h.append(lax.dot_general(w.astype(BF16), vbh,
                                             (((1,), (0,)), ((), ())),
                                             preferred_element_type=F32))
            ctx_b = jnp.concatenate(ctx_h, axis=1)
            part_b = jnp.dot(ctx_b.astype(BF16), wo16,
                             preferred_element_type=F32)
            parts_f32.append(part_b)
            my_part[b] = part_b.astype(BF16)
            for d in (2, 1, 3):
                peer = (me + d) % N_DEV
                slot = _BLOCK_OF_SEND[d] - 1
                rdma = pltpu.make_async_remote_copy(
                    src_ref=my_part.at[b],
                    dst_ref=part_gather.at[slot, b],
                    send_sem=send_p.at[d - 1, b],
                    recv_sem=recv_p.at[slot, b],
                    device_id=(peer,),
                    device_id_type=pl.DeviceIdType.MESH,
                )
                rdma.start()
                p_rdmas.append(rdma)

        for slot in range(N_DEV - 1):
            for b in range(B):
                pltpu.make_async_remote_copy(
                    src_ref=my_part.at[b],
                    dst_ref=part_gather.at[slot, b],
                    send_sem=recv_p.at[slot, b],
                    recv_sem=recv_p.at[slot, b],
                    device_id=((me + 1) % N_DEV,),
                    device_id_type=pl.DeviceIdType.MESH,
                ).wait_recv()

        acc = jnp.stack(parts_f32)
        for j in range(N_DEV - 1):
            acc = acc + part_gather[j].astype(F32)
        out_ref[...] = acc

        for rdma in kv_rdmas + p_rdmas:
            rdma.wait_send()

    return pl.pallas_call(
        body,
        out_shape=jax.ShapeDtypeStruct((B, Sq, DOUT), F32),
        in_specs=[pl.BlockSpec(memory_space=pltpu.VMEM)] * 5,
        out_specs=pl.BlockSpec(memory_space=pltpu.VMEM),
        scratch_shapes=[
            pltpu.VMEM((B, SKV_LOC, Hq * Dh), BF16),
            pltpu.VMEM((B, SKV_LOC, Hq * Dh), BF16),
            pltpu.VMEM((B, SKV, HD_LOC), BF16),
            pltpu.VMEM((B, SKV, HD_LOC), BF16),
            pltpu.VMEM((B, Sq, DOUT), BF16),
            pltpu.VMEM((N_DEV - 1, B, Sq, DOUT), BF16),
            pltpu.SemaphoreType.DMA((N_DEV - 1,)),
            pltpu.SemaphoreType.DMA((N_DEV - 1,)),
            pltpu.SemaphoreType.DMA((N_DEV - 1, B)),
            pltpu.SemaphoreType.DMA((N_DEV - 1,)),
            pltpu.SemaphoreType.DMA((N_DEV - 1,)),
            pltpu.SemaphoreType.DMA((N_DEV - 1, B)),
        ],
        compiler_params=_CompilerParams(collective_id=0),
    )(x, Wq, K_ext, V_ext, Wo)


# device time: 40703 ns/iter; 2.2287x vs baseline; 1.0631x over previous
import jax
import jax.numpy as jnp
from jax import lax
from jax.experimental import pallas as pl
from jax.experimental.pallas import tpu as pltpu

N_DEV = 4
B, Sq, D = 2, 256, 512
Hq, Dh = 16, 64
H_LOC = Hq // N_DEV
HD_LOC = H_LOC * Dh
SKV_LOC = 256
SKV = SKV_LOC * N_DEV
DOUT = 512
BLK = 64
SCALE = 0.125
NEG = -1e9
BF16 = jnp.bfloat16
F32 = jnp.float32

_OFFSET_OF_BLOCK = (0, 1, 3, 2)
_BLOCK_OF_SEND = {1: 2, 2: 3, 3: 1}

_CompilerParams = getattr(pltpu, "CompilerParams", None) or pltpu.TPUCompilerParams


def kernel(x, Wq, K_ext, V_ext, Wo):
    x16 = x.reshape(B * Sq, D).astype(BF16)
    wq16 = Wq.astype(BF16)
    k16 = K_ext.reshape(B, SKV_LOC, Hq * Dh).astype(BF16)
    v16 = V_ext.reshape(B, SKV_LOC, Hq * Dh).astype(BF16)
    wo16 = Wo.astype(BF16)

    def body(x_ref, wq_ref, k_ref, v_ref, wo_ref, out_ref,
             k_all, v_all, scores, my_part, part_gather,
             send_k, send_v, send_p, recv_k, recv_v, recv_p):
        me = lax.axis_index("i")

        barrier = pltpu.get_barrier_semaphore()
        for d in range(1, N_DEV):
            pl.semaphore_signal(
                barrier, inc=1,
                device_id=((me + d) % N_DEV,),
                device_id_type=pl.DeviceIdType.MESH,
            )
        pl.semaphore_wait(barrier, N_DEV - 1)

        kv_rdmas = []
        for d in (2, 1, 3):
            peer = (me + d) % N_DEV
            blk = _BLOCK_OF_SEND[d]
            for (flat, gathered, ssem, rsem) in (
                (k_ref, k_all, send_k, recv_k),
                (v_ref, v_all, send_v, recv_v),
            ):
                rdma = pltpu.make_async_remote_copy(
                    src_ref=flat.at[:, :, pl.ds(peer * HD_LOC, HD_LOC)],
                    dst_ref=gathered.at[:, pl.ds(blk * SKV_LOC, SKV_LOC), :],
                    send_sem=ssem.at[d - 1],
                    recv_sem=rsem.at[blk - 1],
                    device_id=(peer,),
                    device_id_type=pl.DeviceIdType.MESH,
                )
                rdma.start()
                kv_rdmas.append(rdma)

        k_all[:, 0:SKV_LOC, :] = k_ref[:, :, pl.ds(me * HD_LOC, HD_LOC)]
        v_all[:, 0:SKV_LOC, :] = v_ref[:, :, pl.ds(me * HD_LOC, HD_LOC)]

        q = jnp.dot(x_ref[...], wq_ref[...],
                    preferred_element_type=F32).astype(BF16)

        qb = lax.broadcasted_iota(jnp.int32, (Sq, SKV_LOC), 0) // BLK
        jb = lax.broadcasted_iota(jnp.int32, (Sq, SKV_LOC), 1) // BLK

        def score_block(blk):
            src = (me + _OFFSET_OF_BLOCK[blk]) % N_DEV
            kbb = src * (SKV_LOC // BLK) + jb
            mask = (qb == kbb) | (kbb == 0) | (((qb + kbb) % 3) == 0)
            for b in range(B):
                kcb = k_all[b, pl.ds(blk * SKV_LOC, SKV_LOC), :]
                for h in range(H_LOC):
                    qbh = lax.slice(q, (b * Sq, h * Dh),
                                    ((b + 1) * Sq, (h + 1) * Dh))
                    kbh = lax.slice(kcb, (0, h * Dh), (SKV_LOC, (h + 1) * Dh))
                    s = lax.dot_general(qbh, kbh, (((1,), (1,)), ((), ())),
                                        preferred_element_type=F32)
                    scores[b, h, :, pl.ds(blk * SKV_LOC, SKV_LOC)] = (
                        jnp.where(mask, s * SCALE, jnp.float32(NEG)))

        score_block(0)
        for blk in (1, 2, 3):
            for (gathered, rsem) in ((k_all, recv_k), (v_all, recv_v)):
                pltpu.make_async_remote_copy(
                    src_ref=gathered.at[:, pl.ds(blk * SKV_LOC, SKV_LOC), :],
                    dst_ref=gathered.at[:, pl.ds(blk * SKV_LOC, SKV_LOC), :],
                    send_sem=rsem.at[blk - 1],
                    recv_sem=rsem.at[blk - 1],
                    device_id=((me + _OFFSET_OF_BLOCK[blk]) % N_DEV,),
                    device_id_type=pl.DeviceIdType.MESH,
                ).wait_recv()
            score_block(blk)

        p_rdmas = []
        parts_f32 = []
        for b in range(B):
            vb_all = v_all[b]
            ctx_h = []
            for h in range(H_LOC):
                s = scores[b, h]
                m = jnp.max(s, axis=1, keepdims=True)
                w = jnp.exp(s - m)
                w = w / jnp.sum(w, axis=1, keepdims=True)
                vbh = lax.slice(vb_all, (0, h * Dh), (SKV, (h + 1) * Dh))
                ctx_h.append(lax.dot_general(w.astype(BF16), vbh,
                                             (((1,), (0,)), ((), ())),
                                             preferred_element_type=F32))
            ctx_b = jnp.concatenate(ctx_h, axis=1)
            part_b = jnp.dot(ctx_b.astype(BF16), wo_ref[...],
                             preferred_element_type=F32)
            parts_f32.append(part_b)
            my_part[b] = part_b.astype(BF16)
            for d in (2, 1, 3):
                peer = (me + d) % N_DEV
                slot = _BLOCK_OF_SEND[d] - 1
                rdma = pltpu.make_async_remote_copy(
                    src_ref=my_part.at[b],
                    dst_ref=part_gather.at[slot, b],
                    send_sem=send_p.at[d - 1, b],
                    recv_sem=recv_p.at[slot, b],
                    device_id=(peer,),
                    device_id_type=pl.DeviceIdType.MESH,
                )
                rdma.start()
                p_rdmas.append(rdma)

        for slot in range(N_DEV - 1):
            for b in range(B):
                pltpu.make_async_remote_copy(
                    src_ref=my_part.at[b],
                    dst_ref=part_gather.at[slot, b],
                    send_sem=recv_p.at[slot, b],
                    recv_sem=recv_p.at[slot, b],
                    device_id=((me + 1) % N_DEV,),
                    device_id_type=pl.DeviceIdType.MESH,
                ).wait_recv()

        acc = jnp.stack(parts_f32)
        for j in range(N_DEV - 1):
            acc = acc + part_gather[j].astype(F32)
        out_ref[...] = acc

        for rdma in kv_rdmas + p_rdmas:
            rdma.wait_send()

    return pl.pallas_call(
        body,
        out_shape=jax.ShapeDtypeStruct((B, Sq, DOUT), F32),
        in_specs=[pl.BlockSpec(memory_space=pltpu.VMEM)] * 5,
        out_specs=pl.BlockSpec(memory_space=pltpu.VMEM),
        scratch_shapes=[
            pltpu.VMEM((B, SKV, HD_LOC), BF16),
            pltpu.VMEM((B, SKV, HD_LOC), BF16),
            pltpu.VMEM((B, H_LOC, Sq, SKV), F32),
            pltpu.VMEM((B, Sq, DOUT), BF16),
            pltpu.VMEM((N_DEV - 1, B, Sq, DOUT), BF16),
            pltpu.SemaphoreType.DMA((N_DEV - 1,)),
            pltpu.SemaphoreType.DMA((N_DEV - 1,)),
            pltpu.SemaphoreType.DMA((N_DEV - 1, B)),
            pltpu.SemaphoreType.DMA((N_DEV - 1,)),
            pltpu.SemaphoreType.DMA((N_DEV - 1,)),
            pltpu.SemaphoreType.DMA((N_DEV - 1, B)),
        ],
        compiler_params=_CompilerParams(collective_id=0),
    )(x16, wq16, k16, v16, wo16)


# device time: 40388 ns/iter; 2.2460x vs baseline; 1.0078x over previous
import jax
import jax.numpy as jnp
from jax import lax
from jax.experimental import pallas as pl
from jax.experimental.pallas import tpu as pltpu

N_DEV = 4
B, Sq, D = 2, 256, 512
Hq, Dh = 16, 64
H_LOC = Hq // N_DEV
HD_LOC = H_LOC * Dh
SKV_LOC = 256
SKV = SKV_LOC * N_DEV
DOUT = 512
BLK = 64
SCALE = 0.125
NEG = -1e9
RQ = 128
BF16 = jnp.bfloat16
F32 = jnp.float32

_OFFSET_OF_BLOCK = (0, 1, 3, 2)
_BLOCK_OF_SEND = {1: 2, 2: 3, 3: 1}

_CompilerParams = getattr(pltpu, "CompilerParams", None) or pltpu.TPUCompilerParams


def kernel(x, Wq, K_ext, V_ext, Wo):
    x16 = x.reshape(B * Sq, D).astype(BF16)
    wq16 = Wq.astype(BF16)
    k16 = K_ext.reshape(B, SKV_LOC, Hq * Dh).astype(BF16)
    v16 = V_ext.reshape(B, SKV_LOC, Hq * Dh).astype(BF16)
    wo16 = Wo.astype(BF16)

    def body(x_ref, wq_ref, k_ref, v_ref, wo_ref, out_ref,
             k_all, v_all, scores, my_part, part_gather,
             send_k, send_v, send_p, recv_k, recv_v, recv_p):
        me = lax.axis_index("i")

        barrier = pltpu.get_barrier_semaphore()
        for d in range(1, N_DEV):
            pl.semaphore_signal(
                barrier, inc=1,
                device_id=((me + d) % N_DEV,),
                device_id_type=pl.DeviceIdType.MESH,
            )
        pl.semaphore_wait(barrier, N_DEV - 1)

        kv_rdmas = []
        for d in (2, 1, 3):
            peer = (me + d) % N_DEV
            blk = _BLOCK_OF_SEND[d]
            for (flat, gathered, ssem, rsem) in (
                (k_ref, k_all, send_k, recv_k),
                (v_ref, v_all, send_v, recv_v),
            ):
                rdma = pltpu.make_async_remote_copy(
                    src_ref=flat.at[:, :, pl.ds(peer * HD_LOC, HD_LOC)],
                    dst_ref=gathered.at[:, pl.ds(blk * SKV_LOC, SKV_LOC), :],
                    send_sem=ssem.at[d - 1],
                    recv_sem=rsem.at[blk - 1],
                    device_id=(peer,),
                    device_id_type=pl.DeviceIdType.MESH,
                )
                rdma.start()
                kv_rdmas.append(rdma)

        k_all[:, 0:SKV_LOC, :] = k_ref[:, :, pl.ds(me * HD_LOC, HD_LOC)]
        v_all[:, 0:SKV_LOC, :] = v_ref[:, :, pl.ds(me * HD_LOC, HD_LOC)]

        q = jnp.dot(x_ref[...], wq_ref[...],
                    preferred_element_type=F32).astype(BF16)

        qb = lax.broadcasted_iota(jnp.int32, (Sq, SKV_LOC), 0) // BLK
        jb = lax.broadcasted_iota(jnp.int32, (Sq, SKV_LOC), 1) // BLK

        def score_block(blk):
            src = (me + _OFFSET_OF_BLOCK[blk]) % N_DEV
            kbb = src * (SKV_LOC // BLK) + jb
            mask = (qb == kbb) | (kbb == 0) | (((qb + kbb) % 3) == 0)
            for b in range(B):
                kcb = k_all[b, pl.ds(blk * SKV_LOC, SKV_LOC), :]
                for h in range(H_LOC):
                    qbh = lax.slice(q, (b * Sq, h * Dh),
                                    ((b + 1) * Sq, (h + 1) * Dh))
                    kbh = lax.slice(kcb, (0, h * Dh), (SKV_LOC, (h + 1) * Dh))
                    s = lax.dot_general(qbh, kbh, (((1,), (1,)), ((), ())),
                                        preferred_element_type=F32)
                    scores[b, h, :, pl.ds(blk * SKV_LOC, SKV_LOC)] = (
                        jnp.where(mask, s * SCALE, jnp.float32(NEG)))

        score_block(0)
        for blk in (1, 2, 3):
            for (gathered, rsem) in ((k_all, recv_k), (v_all, recv_v)):
                pltpu.make_async_remote_copy(
                    src_ref=gathered.at[:, pl.ds(blk * SKV_LOC, SKV_LOC), :],
                    dst_ref=gathered.at[:, pl.ds(blk * SKV_LOC, SKV_LOC), :],
                    send_sem=rsem.at[blk - 1],
                    recv_sem=rsem.at[blk - 1],
                    device_id=((me + _OFFSET_OF_BLOCK[blk]) % N_DEV,),
                    device_id_type=pl.DeviceIdType.MESH,
                ).wait_recv()
            score_block(blk)

        p_rdmas = []
        parts_f32 = []
        for b in range(B):
            vb_all = v_all[b]
            for r in range(Sq // RQ):
                ctx_h = []
                for h in range(H_LOC):
                    s = scores[b, h, pl.ds(r * RQ, RQ), :]
                    m = jnp.max(s, axis=1, keepdims=True)
                    w = jnp.exp(s - m)
                    w = w / jnp.sum(w, axis=1, keepdims=True)
                    vbh = lax.slice(vb_all, (0, h * Dh), (SKV, (h + 1) * Dh))
                    ctx_h.append(lax.dot_general(w.astype(BF16), vbh,
                                                 (((1,), (0,)), ((), ())),
                                                 preferred_element_type=F32))
                ctx_p = jnp.concatenate(ctx_h, axis=1)
                part_p = jnp.dot(ctx_p.astype(BF16), wo_ref[...],
                                 preferred_element_type=F32)
                parts_f32.append(part_p)
                my_part[b, pl.ds(r * RQ, RQ), :] = part_p.astype(BF16)
                for d in (2, 1, 3):
                    peer = (me + d) % N_DEV
                    slot = _BLOCK_OF_SEND[d] - 1
                    rdma = pltpu.make_async_remote_copy(
                        src_ref=my_part.at[b, pl.ds(r * RQ, RQ), :],
                        dst_ref=part_gather.at[slot, b, pl.ds(r * RQ, RQ), :],
                        send_sem=send_p.at[d - 1, b, r],
                        recv_sem=recv_p.at[slot, b, r],
                        device_id=(peer,),
                        device_id_type=pl.DeviceIdType.MESH,
                    )
                    rdma.start()
                    p_rdmas.append(rdma)

        for slot in range(N_DEV - 1):
            for b in range(B):
                for r in range(Sq // RQ):
                    pltpu.make_async_remote_copy(
                        src_ref=my_part.at[b, pl.ds(r * RQ, RQ), :],
                        dst_ref=part_gather.at[slot, b, pl.ds(r * RQ, RQ), :],
                        send_sem=recv_p.at[slot, b, r],
                        recv_sem=recv_p.at[slot, b, r],
                        device_id=((me + 1) % N_DEV,),
                        device_id_type=pl.DeviceIdType.MESH,
                    ).wait_recv()

        acc = jnp.stack([jnp.concatenate(parts_f32[b * (Sq // RQ):
                                                   (b + 1) * (Sq // RQ)],
                                         axis=0) for b in range(B)])
        for j in range(N_DEV - 1):
            acc = acc + part_gather[j].astype(F32)
        out_ref[...] = acc

        for rdma in kv_rdmas + p_rdmas:
            rdma.wait_send()

    return pl.pallas_call(
        body,
        out_shape=jax.ShapeDtypeStruct((B, Sq, DOUT), F32),
        in_specs=[pl.BlockSpec(memory_space=pltpu.VMEM)] * 5,
        out_specs=pl.BlockSpec(memory_space=pltpu.VMEM),
        scratch_shapes=[
            pltpu.VMEM((B, SKV, HD_LOC), BF16),
            pltpu.VMEM((B, SKV, HD_LOC), BF16),
            pltpu.VMEM((B, H_LOC, Sq, SKV), F32),
            pltpu.VMEM((B, Sq, DOUT), BF16),
            pltpu.VMEM((N_DEV - 1, B, Sq, DOUT), BF16),
            pltpu.SemaphoreType.DMA((N_DEV - 1,)),
            pltpu.SemaphoreType.DMA((N_DEV - 1,)),
            pltpu.SemaphoreType.DMA((N_DEV - 1, B, Sq // RQ)),
            pltpu.SemaphoreType.DMA((N_DEV - 1,)),
            pltpu.SemaphoreType.DMA((N_DEV - 1,)),
            pltpu.SemaphoreType.DMA((N_DEV - 1, B, Sq // RQ)),
        ],
        compiler_params=_CompilerParams(collective_id=0),
    )(x16, wq16, k16, v16, wo16)


# device time: 36071 ns/iter; 2.5148x vs baseline; 1.1197x over previous
import jax
import jax.numpy as jnp
from jax import lax
from jax.experimental import pallas as pl
from jax.experimental.pallas import tpu as pltpu

N_DEV = 4
B, Sq, D = 2, 256, 512
Hq, Dh = 16, 64
H_LOC = Hq // N_DEV
HD_LOC = H_LOC * Dh
SKV_LOC = 256
SKV = SKV_LOC * N_DEV
DOUT = 512
BLK = 64
SCALE = 0.125
NEG = -1e9
RQ = 128
BF16 = jnp.bfloat16
F32 = jnp.float32
FP8 = jnp.float8_e4m3fn

_OFFSET_OF_BLOCK = (0, 1, 3, 2)
_BLOCK_OF_SEND = {1: 2, 2: 3, 3: 1}

_CompilerParams = getattr(pltpu, "CompilerParams", None) or pltpu.TPUCompilerParams


def kernel(x, Wq, K_ext, V_ext, Wo):
    k16 = K_ext.reshape(B, SKV_LOC, Hq * Dh).astype(FP8)
    v16 = V_ext.reshape(B, SKV_LOC, Hq * Dh).astype(BF16)

    def body(x_ref, wq_ref, k_ref, v_ref, wo_ref, out_ref,
             k_all, v_all, scores, my_part, part_gather,
             send_k, send_v, send_p, recv_k, recv_v, recv_p):
        me = lax.axis_index("i")

        barrier = pltpu.get_barrier_semaphore()
        for d in range(1, N_DEV):
            pl.semaphore_signal(
                barrier, inc=1,
                device_id=((me + d) % N_DEV,),
                device_id_type=pl.DeviceIdType.MESH,
            )
        k_all[:, 0:SKV_LOC, :] = k_ref[:, :, pl.ds(me * HD_LOC, HD_LOC)]
        v_all[:, 0:SKV_LOC, :] = v_ref[:, :, pl.ds(me * HD_LOC, HD_LOC)]
        q = jnp.dot(x_ref[...].reshape(B * Sq, D).astype(BF16),
                    wq_ref[...].astype(BF16),
                    preferred_element_type=F32).astype(BF16)

        pl.semaphore_wait(barrier, N_DEV - 1)

        kv_rdmas = []
        for d in (2, 1, 3):
            peer = (me + d) % N_DEV
            blk = _BLOCK_OF_SEND[d]
            for (flat, gathered, ssem, rsem) in (
                (k_ref, k_all, send_k, recv_k),
                (v_ref, v_all, send_v, recv_v),
            ):
                rdma = pltpu.make_async_remote_copy(
                    src_ref=flat.at[:, :, pl.ds(peer * HD_LOC, HD_LOC)],
                    dst_ref=gathered.at[:, pl.ds(blk * SKV_LOC, SKV_LOC), :],
                    send_sem=ssem.at[d - 1],
                    recv_sem=rsem.at[blk - 1],
                    device_id=(peer,),
                    device_id_type=pl.DeviceIdType.MESH,
                )
                rdma.start()
                kv_rdmas.append(rdma)

        qb = lax.broadcasted_iota(jnp.int32, (Sq, SKV_LOC), 0) // BLK
        jb = lax.broadcasted_iota(jnp.int32, (Sq, SKV_LOC), 1) // BLK

        def score_block(blk):
            src = (me + _OFFSET_OF_BLOCK[blk]) % N_DEV
            kbb = src * (SKV_LOC // BLK) + jb
            mask = (qb == kbb) | (kbb == 0) | (((qb + kbb) % 3) == 0)
            for b in range(B):
                kcb = k_all[b, pl.ds(blk * SKV_LOC, SKV_LOC), :].astype(BF16)
                for h in range(H_LOC):
                    qbh = lax.slice(q, (b * Sq, h * Dh),
                                    ((b + 1) * Sq, (h + 1) * Dh))
                    kbh = lax.slice(kcb, (0, h * Dh), (SKV_LOC, (h + 1) * Dh))
                    s = lax.dot_general(qbh, kbh, (((1,), (1,)), ((), ())),
                                        preferred_element_type=F32)
                    scores[b, h, :, pl.ds(blk * SKV_LOC, SKV_LOC)] = (
                        jnp.where(mask, s * SCALE, jnp.float32(NEG)))

        score_block(0)
        for blk in (1, 2, 3):
            for (gathered, rsem) in ((k_all, recv_k), (v_all, recv_v)):
                pltpu.make_async_remote_copy(
                    src_ref=gathered.at[:, pl.ds(blk * SKV_LOC, SKV_LOC), :],
                    dst_ref=gathered.at[:, pl.ds(blk * SKV_LOC, SKV_LOC), :],
                    send_sem=rsem.at[blk - 1],
                    recv_sem=rsem.at[blk - 1],
                    device_id=((me + _OFFSET_OF_BLOCK[blk]) % N_DEV,),
                    device_id_type=pl.DeviceIdType.MESH,
                ).wait_recv()
            score_block(blk)

        wo16 = wo_ref[...].astype(BF16)
        p_rdmas = []
        parts_f32 = []
        for b in range(B):
            vb_all = v_all[b]
            for r in range(Sq // RQ):
                ctx_h = []
                for h in range(H_LOC):
                    s = scores[b, h, pl.ds(r * RQ, RQ), :]
                    m = jnp.max(s, axis=1, keepdims=True)
                    w = jnp.exp(s - m)
                    w = w / jnp.sum(w, axis=1, keepdims=True)
                    vbh = lax.slice(vb_all, (0, h * Dh), (SKV, (h + 1) * Dh))
                    ctx_h.append(lax.dot_general(w.astype(BF16), vbh,
                                                 (((1,), (0,)), ((), ())),
                                                 preferred_element_type=F32))
                ctx_p = jnp.concatenate(ctx_h, axis=1)
                part_p = jnp.dot(ctx_p.astype(BF16), wo16,
                                 preferred_element_type=F32)
                parts_f32.append(part_p)
                my_part[b, pl.ds(r * RQ, RQ), :] = part_p.astype(BF16)
                for d in (2, 1, 3):
                    peer = (me + d) % N_DEV
                    slot = _BLOCK_OF_SEND[d] - 1
                    rdma = pltpu.make_async_remote_copy(
                        src_ref=my_part.at[b, pl.ds(r * RQ, RQ), :],
                        dst_ref=part_gather.at[slot, b, pl.ds(r * RQ, RQ), :],
                        send_sem=send_p.at[d - 1, b, r],
                        recv_sem=recv_p.at[slot, b, r],
                        device_id=(peer,),
                        device_id_type=pl.DeviceIdType.MESH,
                    )
                    rdma.start()
                    p_rdmas.append(rdma)

        for slot in range(N_DEV - 1):
            for b in range(B):
                for r in range(Sq // RQ):
                    pltpu.make_async_remote_copy(
                        src_ref=my_part.at[b, pl.ds(r * RQ, RQ), :],
                        dst_ref=part_gather.at[slot, b, pl.ds(r * RQ, RQ), :],
                        send_sem=recv_p.at[slot, b, r],
                        recv_sem=recv_p.at[slot, b, r],
                        device_id=((me + 1) % N_DEV,),
                        device_id_type=pl.DeviceIdType.MESH,
                    ).wait_recv()

        acc = jnp.stack([jnp.concatenate(parts_f32[b * (Sq // RQ):
                                                   (b + 1) * (Sq // RQ)],
                                         axis=0) for b in range(B)])
        for j in range(N_DEV - 1):
            acc = acc + part_gather[j].astype(F32)
        out_ref[...] = acc

        for rdma in kv_rdmas + p_rdmas:
            rdma.wait_send()

    return pl.pallas_call(
        body,
        out_shape=jax.ShapeDtypeStruct((B, Sq, DOUT), F32),
        in_specs=[pl.BlockSpec(memory_space=pltpu.VMEM)] * 5,
        out_specs=pl.BlockSpec(memory_space=pltpu.VMEM),
        scratch_shapes=[
            pltpu.VMEM((B, SKV, HD_LOC), FP8),
            pltpu.VMEM((B, SKV, HD_LOC), BF16),
            pltpu.VMEM((B, H_LOC, Sq, SKV), F32),
            pltpu.VMEM((B, Sq, DOUT), BF16),
            pltpu.VMEM((N_DEV - 1, B, Sq, DOUT), BF16),
            pltpu.SemaphoreType.DMA((N_DEV - 1,)),
            pltpu.SemaphoreType.DMA((N_DEV - 1,)),
            pltpu.SemaphoreType.DMA((N_DEV - 1, B, Sq // RQ)),
            pltpu.SemaphoreType.DMA((N_DEV - 1,)),
            pltpu.SemaphoreType.DMA((N_DEV - 1,)),
            pltpu.SemaphoreType.DMA((N_DEV - 1, B, Sq // RQ)),
        ],
        compiler_params=_CompilerParams(collective_id=0),
    )(x, Wq, k16, v16, Wo)
